# baseline (speedup 1.0000x reference)
import jax
import jax.numpy as jnp
from jax.experimental import pallas as pl
from jax.experimental.pallas import tpu as pltpu

NEG = -1e10
F32 = jnp.float32
N_DOC, MS, W, BH, H, ED = 8, 16, 10, 768, 1024, 20
NS = N_DOC * MS
SD = 2 * BH + BH + ED
F_, R, D = 64, 36, 1024


def _prologue_kernel(se_ref, cont_ref, width_ref, doc_ref, img_ref,
                     aw1_ref, ab1_ref, aw2_ref, ab2_ref, wemb_ref, w1_any,
                     x_ref, xb_ref, a_ref, b_ref, sg_ref, w1c_ref,
                     w1_vmem, sem):
    cp = pltpu.make_async_copy(w1_any, w1_vmem, sem)
    cp.start()
    width = width_ref[...]
    score_cols = []
    for t in range(W):
        ct = cont_ref[:, t, :]
        h = jnp.maximum(
            jnp.dot(ct, aw1_ref[...], preferred_element_type=F32) + ab1_ref[...],
            0.0)
        score_cols.append(jnp.dot(h, aw2_ref[...], preferred_element_type=F32)
                          + ab2_ref[...])
    scores = jnp.concatenate(score_cols, axis=1)
    tok = jax.lax.broadcasted_iota(jnp.int32, (NS, W), 1).astype(F32)
    scores = jnp.where(tok < width, scores, NEG)
    m = jnp.max(scores, axis=1, keepdims=True)
    e = jnp.exp(scores - m)
    attn = e / jnp.sum(e, axis=1, keepdims=True)
    weighted = jnp.zeros((NS, BH), F32)
    for t in range(W):
        weighted = weighted + attn[:, t:t + 1] * cont_ref[:, t, :]
    wclip = jnp.minimum(width, 4.0)
    wfeat = jnp.zeros((NS, ED), F32)
    for k in range(5):
        wfeat = wfeat + jnp.where(wclip == k, 1.0, 0.0) * wemb_ref[k:k + 1, :]
    x = jnp.concatenate([se_ref[...], weighted, wfeat], axis=1)
    x_ref[...] = x
    xb_ref[...] = x.astype(jnp.bfloat16)

    doc = doc_ref[...]
    sg_cols = []
    for v in range(N_DOC):
        imgv = img_ref[v * R:(v + 1) * R, :]
        att = jax.lax.dot_general(doc, imgv, (((1,), (1,)), ((), ())),
                                  preferred_element_type=F32)
        att = att.reshape(N_DOC, F_, R)
        att = jnp.where(att != 0.0, att, NEG)
        m1 = jnp.max(att, axis=2, keepdims=True)
        e1 = jnp.exp(att - m1)
        aw1 = e1 / jnp.sum(e1, axis=2, keepdims=True)
        s1 = jnp.sum(aw1 * att, axis=(1, 2))
        m2 = jnp.max(att, axis=1, keepdims=True)
        e2 = jnp.exp(att - m2)
        aw2 = e2 / jnp.sum(e2, axis=1, keepdims=True)
        s2 = jnp.sum(aw2 * att, axis=(1, 2))
        sg_cols.append((s1 + s2).reshape(N_DOC, 1))
    sg_ref[...] = jnp.concatenate(sg_cols, axis=1)

    cp.wait()
    a_ref[...] = jnp.dot(x, w1_vmem[0:SD, :], preferred_element_type=F32)
    xs = jnp.concatenate([jnp.zeros((NS, 4), F32), x], axis=1)
    b_ref[...] = jnp.dot(xs, w1_vmem[SD - 4:2 * SD, :],
                         preferred_element_type=F32)
    w1c_ref[...] = w1_vmem[2 * SD:3 * SD, :].astype(jnp.bfloat16)


def _pair_kernel(xf_ref, xb_ref, w1c_ref, w2_ref, w3_ref, a_ref, b_ref,
                 b1_ref, b2_ref, b3_ref, out_ref):
    v = pl.program_id(0)
    lane16 = jax.lax.broadcasted_iota(jnp.int32, (1, MS), 1)

    def body(j, ts):
        q = v * MS + j
        xq = xf_ref[pl.ds(q, 1), :].astype(jnp.bfloat16)
        z = jnp.dot(xb_ref[...] * xq, w1c_ref[...],
                    preferred_element_type=F32)
        h1 = jnp.maximum(z + a_ref[...] + b_ref[pl.ds(q, 1), :]
                         + b1_ref[...], 0.0)
        h2 = jnp.maximum(jnp.dot(h1.astype(jnp.bfloat16), w2_ref[...],
                                 preferred_element_type=F32) + b2_ref[...], 0.0)
        col = jnp.dot(h2, w3_ref[...], preferred_element_type=F32) \
            + b3_ref[...]
        onehot = jnp.where(lane16 == j, 1.0, 0.0)
        return ts + col * onehot

    ts = jax.lax.fori_loop(0, MS, body, jnp.zeros((NS, MS), F32), unroll=4)
    rm = jnp.mean(ts, axis=1, keepdims=True)
    max1 = jnp.max(rm.reshape(N_DOC, MS, 1), axis=1)
    cm = jnp.mean(ts.reshape(N_DOC, MS, MS), axis=1)
    max2 = jnp.max(cm, axis=1, keepdims=True)
    both = jnp.concatenate([max1, max2], axis=0)
    out_ref[0] = jnp.broadcast_to(both, (MS, 128))


def _loss_kernel(sg_ref, bout_ref, out_ref):
    red = jnp.max(bout_ref[...], axis=2)
    scT = 0.5 * (red[:, 0:N_DOC] + red[:, N_DOC:2 * N_DOC])
    sc = scT.T
    sg = sg_ref[...]

    def rowsoftmax(x):
        m = jnp.max(x, axis=1, keepdims=True)
        e = jnp.exp(x - m)
        return e / jnp.sum(e, axis=1, keepdims=True)

    mg = rowsoftmax(sg)
    mgT = rowsoftmax(sg.T)
    mc = rowsoftmax(sc)
    t1 = jnp.log(jnp.sum(mg * mc, axis=1, keepdims=True))
    t2 = jnp.log(jnp.sum(mgT * mc, axis=1, keepdims=True))
    loss = -(jnp.sum(t1) + jnp.sum(t2)) / float(N_DOC)
    out_ref[...] = jnp.broadcast_to(loss, (1, 1))


def kernel(doc_embeddings, image_embeddings, text_mask, image_mask,
           start_end_embeddings, continuous_embeddings, width, span_mask,
           attn_w1, attn_b1, attn_w2, attn_b2, width_emb,
           pw_w1, pw_b1, pw_w2, pw_b2, pw_w3, pw_b3):
    se = start_end_embeddings.reshape(NS, 2 * BH)
    cont = continuous_embeddings.reshape(NS, W, BH)
    wid = width.astype(F32).reshape(NS, 1)
    docR = doc_embeddings.reshape(N_DOC * F_, D)
    imgR = image_embeddings.reshape(N_DOC * R, D)
    ab1 = attn_b1.reshape(1, H)
    ab2 = attn_b2.reshape(1, 1)
    b1 = pw_b1.reshape(1, H)
    b2 = pw_b2.reshape(1, H)
    b3 = pw_b3.reshape(1, 1)

    x, xb, a, b, sg, w1c = pl.pallas_call(
        _prologue_kernel,
        out_shape=[
            jax.ShapeDtypeStruct((NS, SD), F32),
            jax.ShapeDtypeStruct((NS, SD), jnp.bfloat16),
            jax.ShapeDtypeStruct((NS, H), F32),
            jax.ShapeDtypeStruct((NS, H), F32),
            jax.ShapeDtypeStruct((N_DOC, N_DOC), F32),
            jax.ShapeDtypeStruct((SD, H), jnp.bfloat16),
        ],
        in_specs=[pl.BlockSpec()] * 10 + [pl.BlockSpec(memory_space=pl.ANY)],
        scratch_shapes=[pltpu.VMEM((3 * SD, H), F32), pltpu.SemaphoreType.DMA],
        compiler_params=pltpu.CompilerParams(
            vmem_limit_bytes=56 * 1024 * 1024),
        name="coref_prologue",
    )(se, cont, wid, docR, imgR, attn_w1, ab1, attn_w2, ab2, width_emb, pw_w1)

    bout = pl.pallas_call(
        _pair_kernel,
        out_shape=jax.ShapeDtypeStruct((N_DOC, MS, 128), F32),
        grid=(8,),
        in_specs=[
            pl.BlockSpec((NS, SD), lambda v: (0, 0)),
            pl.BlockSpec((NS, SD), lambda v: (0, 0)),
            pl.BlockSpec((SD, H), lambda v: (0, 0)),
            pl.BlockSpec((H, H), lambda v: (0, 0)),
            pl.BlockSpec((H, 1), lambda v: (0, 0)),
            pl.BlockSpec((NS, H), lambda v: (0, 0)),
            pl.BlockSpec((NS, H), lambda v: (0, 0)),
            pl.BlockSpec((1, H), lambda v: (0, 0)),
            pl.BlockSpec((1, H), lambda v: (0, 0)),
            pl.BlockSpec((1, 1), lambda v: (0, 0)),
        ],
        out_specs=pl.BlockSpec((1, MS, 128), lambda v: (v, 0, 0)),
        compiler_params=pltpu.CompilerParams(
            dimension_semantics=("arbitrary",),
            vmem_limit_bytes=56 * 1024 * 1024),
        name="coref_pairs",
    )(x, xb, w1c, pw_w2.astype(jnp.bfloat16), pw_w3, a, b, b1, b2, b3)

    loss = pl.pallas_call(
        _loss_kernel,
        out_shape=jax.ShapeDtypeStruct((1, 1), F32),
        name="coref_loss",
    )(sg, bout)
    return loss.reshape(())


# speedup vs baseline: 1.0669x; 1.0669x over previous
import jax
import jax.numpy as jnp
from jax.experimental import pallas as pl
from jax.experimental.pallas import tpu as pltpu

NEG = -1e10
F32 = jnp.float32
N_DOC, MS, W, BH, H, ED = 8, 16, 10, 768, 1024, 20
NS = N_DOC * MS
SD = 2 * BH + BH + ED
F_, R, D = 64, 36, 1024


def _prologue_kernel(se_ref, cont_ref, width_ref, doc_ref, img_ref,
                     aw1_ref, ab1_ref, aw2_ref, ab2_ref, wemb_ref, w1_any,
                     x_ref, a_ref, b_ref, sg_ref, w1ab_vmem, sem):
    cp = pltpu.make_async_copy(w1_any.at[pl.ds(0, 2 * SD), :], w1ab_vmem, sem)
    cp.start()
    width = width_ref[...]
    score_cols = []
    for t in range(W):
        ct = cont_ref[:, t, :]
        h = jnp.maximum(
            jnp.dot(ct, aw1_ref[...], preferred_element_type=F32) + ab1_ref[...],
            0.0)
        score_cols.append(jnp.dot(h, aw2_ref[...], preferred_element_type=F32)
                          + ab2_ref[...])
    scores = jnp.concatenate(score_cols, axis=1)
    tok = jax.lax.broadcasted_iota(jnp.int32, (NS, W), 1).astype(F32)
    scores = jnp.where(tok < width, scores, NEG)
    m = jnp.max(scores, axis=1, keepdims=True)
    e = jnp.exp(scores - m)
    attn = e / jnp.sum(e, axis=1, keepdims=True)
    weighted = jnp.zeros((NS, BH), F32)
    for t in range(W):
        weighted = weighted + attn[:, t:t + 1] * cont_ref[:, t, :]
    wclip = jnp.minimum(width, 4.0)
    wfeat = jnp.zeros((NS, ED), F32)
    for k in range(5):
        wfeat = wfeat + jnp.where(wclip == k, 1.0, 0.0) * wemb_ref[k:k + 1, :]
    x = jnp.concatenate([se_ref[...], weighted, wfeat], axis=1)
    x_ref[...] = x

    doc = doc_ref[...]
    sg_cols = []
    for v in range(N_DOC):
        imgv = img_ref[v * R:(v + 1) * R, :]
        att = jax.lax.dot_general(doc, imgv, (((1,), (1,)), ((), ())),
                                  preferred_element_type=F32)
        att = att.reshape(N_DOC, F_, R)
        att = jnp.where(att != 0.0, att, NEG)
        m1 = jnp.max(att, axis=2, keepdims=True)
        e1 = jnp.exp(att - m1)
        aw1 = e1 / jnp.sum(e1, axis=2, keepdims=True)
        s1 = jnp.sum(aw1 * att, axis=(1, 2))
        m2 = jnp.max(att, axis=1, keepdims=True)
        e2 = jnp.exp(att - m2)
        aw2 = e2 / jnp.sum(e2, axis=1, keepdims=True)
        s2 = jnp.sum(aw2 * att, axis=(1, 2))
        sg_cols.append((s1 + s2).reshape(N_DOC, 1))
    sg_ref[...] = jnp.concatenate(sg_cols, axis=1)

    cp.wait()
    a_ref[...] = jnp.dot(x, w1ab_vmem[0:SD, :], preferred_element_type=F32)
    xs = jnp.concatenate([jnp.zeros((NS, 4), F32), x], axis=1)
    b_ref[...] = jnp.dot(xs, w1ab_vmem[SD - 4:2 * SD, :],
                         preferred_element_type=F32)


def _pair_kernel(xf_ref, w2_ref, w3_ref, a_ref, b_ref,
                 b1_ref, b2_ref, b3_ref, w1_any, out_ref, w1c_ref, sem):
    v = pl.program_id(0)

    @pl.when(v == 0)
    def _():
        cp = pltpu.make_async_copy(w1_any.at[pl.ds(2 * SD, SD), :],
                                   w1c_ref, sem)
        cp.start()
        cp.wait()

    lane16 = jax.lax.broadcasted_iota(jnp.int32, (1, MS), 1)

    def body(j, ts):
        q = v * MS + j
        xq = xf_ref[pl.ds(q, 1), :]
        z = jnp.dot(xf_ref[...] * xq, w1c_ref[...],
                    preferred_element_type=F32)
        h1 = jnp.maximum(z + a_ref[...] + b_ref[pl.ds(q, 1), :]
                         + b1_ref[...], 0.0)
        h2 = jnp.maximum(jnp.dot(h1, w2_ref[...],
                                 preferred_element_type=F32) + b2_ref[...], 0.0)
        col = jnp.dot(h2, w3_ref[...], preferred_element_type=F32) \
            + b3_ref[...]
        onehot = jnp.where(lane16 == j, 1.0, 0.0)
        return ts + col * onehot

    ts = jax.lax.fori_loop(0, MS, body, jnp.zeros((NS, MS), F32), unroll=8)
    rm = jnp.mean(ts, axis=1, keepdims=True)
    max1 = jnp.max(rm.reshape(N_DOC, MS, 1), axis=1)
    cm = jnp.mean(ts.reshape(N_DOC, MS, MS), axis=1)
    max2 = jnp.max(cm, axis=1, keepdims=True)
    both = jnp.concatenate([max1, max2], axis=0)
    out_ref[0] = jnp.broadcast_to(both, (MS, 128))


def _loss_kernel(sg_ref, bout_ref, out_ref):
    red = jnp.max(bout_ref[...], axis=2)
    scT = 0.5 * (red[:, 0:N_DOC] + red[:, N_DOC:2 * N_DOC])
    sc = scT.T
    sg = sg_ref[...]

    def rowsoftmax(x):
        m = jnp.max(x, axis=1, keepdims=True)
        e = jnp.exp(x - m)
        return e / jnp.sum(e, axis=1, keepdims=True)

    mg = rowsoftmax(sg)
    mgT = rowsoftmax(sg.T)
    mc = rowsoftmax(sc)
    t1 = jnp.log(jnp.sum(mg * mc, axis=1, keepdims=True))
    t2 = jnp.log(jnp.sum(mgT * mc, axis=1, keepdims=True))
    loss = -(jnp.sum(t1) + jnp.sum(t2)) / float(N_DOC)
    out_ref[...] = jnp.broadcast_to(loss, (1, 1))


def kernel(doc_embeddings, image_embeddings, text_mask, image_mask,
           start_end_embeddings, continuous_embeddings, width, span_mask,
           attn_w1, attn_b1, attn_w2, attn_b2, width_emb,
           pw_w1, pw_b1, pw_w2, pw_b2, pw_w3, pw_b3):
    se = start_end_embeddings.reshape(NS, 2 * BH)
    cont = continuous_embeddings.reshape(NS, W, BH)
    wid = width.astype(F32).reshape(NS, 1)
    docR = doc_embeddings.reshape(N_DOC * F_, D)
    imgR = image_embeddings.reshape(N_DOC * R, D)
    ab1 = attn_b1.reshape(1, H)
    ab2 = attn_b2.reshape(1, 1)
    b1 = pw_b1.reshape(1, H)
    b2 = pw_b2.reshape(1, H)
    b3 = pw_b3.reshape(1, 1)

    x, a, b, sg = pl.pallas_call(
        _prologue_kernel,
        out_shape=[
            jax.ShapeDtypeStruct((NS, SD), F32),
            jax.ShapeDtypeStruct((NS, H), F32),
            jax.ShapeDtypeStruct((NS, H), F32),
            jax.ShapeDtypeStruct((N_DOC, N_DOC), F32),
        ],
        in_specs=[pl.BlockSpec()] * 10 + [pl.BlockSpec(memory_space=pl.ANY)],
        scratch_shapes=[pltpu.VMEM((2 * SD, H), F32), pltpu.SemaphoreType.DMA],
        compiler_params=pltpu.CompilerParams(
            vmem_limit_bytes=56 * 1024 * 1024),
        name="coref_prologue",
    )(se, cont, wid, docR, imgR, attn_w1, ab1, attn_w2, ab2, width_emb, pw_w1)

    bout = pl.pallas_call(
        _pair_kernel,
        out_shape=jax.ShapeDtypeStruct((N_DOC, MS, 128), F32),
        grid=(8,),
        in_specs=[
            pl.BlockSpec((NS, SD), lambda v: (0, 0)),
            pl.BlockSpec((H, H), lambda v: (0, 0)),
            pl.BlockSpec((H, 1), lambda v: (0, 0)),
            pl.BlockSpec((NS, H), lambda v: (0, 0)),
            pl.BlockSpec((NS, H), lambda v: (0, 0)),
            pl.BlockSpec((1, H), lambda v: (0, 0)),
            pl.BlockSpec((1, H), lambda v: (0, 0)),
            pl.BlockSpec((1, 1), lambda v: (0, 0)),
            pl.BlockSpec(memory_space=pl.ANY),
        ],
        out_specs=pl.BlockSpec((1, MS, 128), lambda v: (v, 0, 0)),
        scratch_shapes=[pltpu.VMEM((SD, H), F32), pltpu.SemaphoreType.DMA],
        compiler_params=pltpu.CompilerParams(
            dimension_semantics=("arbitrary",),
            vmem_limit_bytes=56 * 1024 * 1024),
        name="coref_pairs",
    )(x, pw_w2, pw_w3, a, b, b1, b2, b3, pw_w1)

    loss = pl.pallas_call(
        _loss_kernel,
        out_shape=jax.ShapeDtypeStruct((1, 1), F32),
        name="coref_loss",
    )(sg, bout)
    return loss.reshape(())


# speedup vs baseline: 1.0735x; 1.0062x over previous
import jax
import jax.numpy as jnp
from jax.experimental import pallas as pl
from jax.experimental.pallas import tpu as pltpu

NEG = -1e10
F32 = jnp.float32
N_DOC, MS, W, BH, H, ED = 8, 16, 10, 768, 1024, 20
NS = N_DOC * MS
SD = 2 * BH + BH + ED
F_, R, D = 64, 36, 1024


def _prologue_kernel(se_ref, cont_ref, width_ref, doc_ref, img_ref,
                     aw1_ref, ab1_ref, aw2_ref, ab2_ref, wemb_ref, w1_any,
                     x_ref, a_ref, b_ref, sg_ref, w1ab_vmem, sem):
    cp = pltpu.make_async_copy(w1_any.at[pl.ds(0, 2 * SD), :], w1ab_vmem, sem)
    cp.start()
    width = width_ref[...]
    score_cols = []
    for t in range(W):
        ct = cont_ref[:, t * BH:(t + 1) * BH]
        h = jnp.maximum(
            jnp.dot(ct, aw1_ref[...], preferred_element_type=F32) + ab1_ref[...],
            0.0)
        score_cols.append(jnp.dot(h, aw2_ref[...], preferred_element_type=F32)
                          + ab2_ref[...])
    scores = jnp.concatenate(score_cols, axis=1)
    tok = jax.lax.broadcasted_iota(jnp.int32, (NS, W), 1).astype(F32)
    scores = jnp.where(tok < width, scores, NEG)
    m = jnp.max(scores, axis=1, keepdims=True)
    e = jnp.exp(scores - m)
    attn = e / jnp.sum(e, axis=1, keepdims=True)
    weighted = jnp.zeros((NS, BH), F32)
    for t in range(W):
        weighted = weighted + attn[:, t:t + 1] * cont_ref[:, t * BH:(t + 1) * BH]
    wclip = jnp.minimum(width, 4.0)
    wfeat = jnp.zeros((NS, ED), F32)
    for k in range(5):
        wfeat = wfeat + jnp.where(wclip == k, 1.0, 0.0) * wemb_ref[k:k + 1, :]
    x = jnp.concatenate([se_ref[...], weighted, wfeat], axis=1)
    x_ref[...] = x

    doc = doc_ref[...]
    sg_cols = []
    for v in range(N_DOC):
        imgv = img_ref[v * R:(v + 1) * R, :]
        att = jax.lax.dot_general(doc, imgv, (((1,), (1,)), ((), ())),
                                  preferred_element_type=F32)
        att = att.reshape(N_DOC, F_, R)
        att = jnp.where(att != 0.0, att, NEG)
        m1 = jnp.max(att, axis=2, keepdims=True)
        e1 = jnp.exp(att - m1)
        aw1 = e1 / jnp.sum(e1, axis=2, keepdims=True)
        s1 = jnp.sum(aw1 * att, axis=(1, 2))
        m2 = jnp.max(att, axis=1, keepdims=True)
        e2 = jnp.exp(att - m2)
        aw2 = e2 / jnp.sum(e2, axis=1, keepdims=True)
        s2 = jnp.sum(aw2 * att, axis=(1, 2))
        sg_cols.append((s1 + s2).reshape(N_DOC, 1))
    sg_ref[...] = jnp.concatenate(sg_cols, axis=1)

    cp.wait()
    a_ref[...] = jnp.dot(x, w1ab_vmem[0:SD, :], preferred_element_type=F32)
    xs = jnp.concatenate([jnp.zeros((NS, 4), F32), x], axis=1)
    b_ref[...] = jnp.dot(xs, w1ab_vmem[SD - 4:2 * SD, :],
                         preferred_element_type=F32)


def _pair_kernel(xf_ref, w2_ref, w3_ref, a_ref, b_ref,
                 b1_ref, b2_ref, b3_ref, sg_ref, w1_any, loss_ref,
                 w1c_ref, sc_ref, sem):
    v = pl.program_id(0)

    @pl.when(v == 0)
    def _():
        cp = pltpu.make_async_copy(w1_any.at[pl.ds(2 * SD, SD), :],
                                   w1c_ref, sem)
        cp.start()
        cp.wait()
        sc_ref[...] = jnp.zeros_like(sc_ref)

    lane16 = jax.lax.broadcasted_iota(jnp.int32, (1, MS), 1)

    def body(j, ts):
        q = v * MS + j
        xq = xf_ref[pl.ds(q, 1), :]
        z = jnp.dot(xf_ref[...] * xq, w1c_ref[...],
                    preferred_element_type=F32)
        h1 = jnp.maximum(z + a_ref[...] + b_ref[pl.ds(q, 1), :]
                         + b1_ref[...], 0.0)
        h2 = jnp.maximum(jnp.dot(h1, w2_ref[...],
                                 preferred_element_type=F32) + b2_ref[...], 0.0)
        col = jnp.dot(h2, w3_ref[...], preferred_element_type=F32) \
            + b3_ref[...]
        onehot = jnp.where(lane16 == j, 1.0, 0.0)
        return ts + col * onehot

    ts = jax.lax.fori_loop(0, MS, body, jnp.zeros((NS, MS), F32), unroll=8)
    rm = jnp.mean(ts, axis=1, keepdims=True)
    max1 = jnp.max(rm.reshape(N_DOC, MS, 1), axis=1)
    cm = jnp.mean(ts.reshape(N_DOC, MS, MS), axis=1)
    max2 = jnp.max(cm, axis=1, keepdims=True)
    both = jnp.concatenate([max1, max2], axis=0)
    onehot_v = jnp.where(
        jax.lax.broadcasted_iota(jnp.int32, (1, 128), 1) == v, 1.0, 0.0)
    sc_ref[...] = sc_ref[...] + both * onehot_v

    @pl.when(v == N_DOC - 1)
    def _():
        sc = 0.5 * (sc_ref[0:N_DOC, 0:N_DOC]
                    + sc_ref[N_DOC:2 * N_DOC, 0:N_DOC])
        sg = sg_ref[...]

        def rowsoftmax(m_):
            mx = jnp.max(m_, axis=1, keepdims=True)
            e = jnp.exp(m_ - mx)
            return e / jnp.sum(e, axis=1, keepdims=True)

        mg = rowsoftmax(sg)
        mgT = rowsoftmax(sg.T)
        mc = rowsoftmax(sc)
        t1 = jnp.log(jnp.sum(mg * mc, axis=1, keepdims=True))
        t2 = jnp.log(jnp.sum(mgT * mc, axis=1, keepdims=True))
        loss = -(jnp.sum(t1) + jnp.sum(t2)) / float(N_DOC)
        loss_ref[...] = jnp.broadcast_to(loss, (1, 1))


def kernel(doc_embeddings, image_embeddings, text_mask, image_mask,
           start_end_embeddings, continuous_embeddings, width, span_mask,
           attn_w1, attn_b1, attn_w2, attn_b2, width_emb,
           pw_w1, pw_b1, pw_w2, pw_b2, pw_w3, pw_b3):
    se = start_end_embeddings.reshape(NS, 2 * BH)
    cont = continuous_embeddings.reshape(NS, W * BH)
    wid = width.astype(F32).reshape(NS, 1)
    docR = doc_embeddings.reshape(N_DOC * F_, D)
    imgR = image_embeddings.reshape(N_DOC * R, D)
    ab1 = attn_b1.reshape(1, H)
    ab2 = attn_b2.reshape(1, 1)
    b1 = pw_b1.reshape(1, H)
    b2 = pw_b2.reshape(1, H)
    b3 = pw_b3.reshape(1, 1)

    x, a, b, sg = pl.pallas_call(
        _prologue_kernel,
        out_shape=[
            jax.ShapeDtypeStruct((NS, SD), F32),
            jax.ShapeDtypeStruct((NS, H), F32),
            jax.ShapeDtypeStruct((NS, H), F32),
            jax.ShapeDtypeStruct((N_DOC, N_DOC), F32),
        ],
        in_specs=[pl.BlockSpec()] * 10 + [pl.BlockSpec(memory_space=pl.ANY)],
        scratch_shapes=[pltpu.VMEM((2 * SD, H), F32), pltpu.SemaphoreType.DMA],
        compiler_params=pltpu.CompilerParams(
            vmem_limit_bytes=56 * 1024 * 1024),
        name="coref_prologue",
    )(se, cont, wid, docR, imgR, attn_w1, ab1, attn_w2, ab2, width_emb, pw_w1)

    loss = pl.pallas_call(
        _pair_kernel,
        out_shape=jax.ShapeDtypeStruct((1, 1), F32),
        grid=(8,),
        in_specs=[
            pl.BlockSpec((NS, SD), lambda v: (0, 0)),
            pl.BlockSpec((H, H), lambda v: (0, 0)),
            pl.BlockSpec((H, 1), lambda v: (0, 0)),
            pl.BlockSpec((NS, H), lambda v: (0, 0)),
            pl.BlockSpec((NS, H), lambda v: (0, 0)),
            pl.BlockSpec((1, H), lambda v: (0, 0)),
            pl.BlockSpec((1, H), lambda v: (0, 0)),
            pl.BlockSpec((1, 1), lambda v: (0, 0)),
            pl.BlockSpec((N_DOC, N_DOC), lambda v: (0, 0)),
            pl.BlockSpec(memory_space=pl.ANY),
        ],
        out_specs=pl.BlockSpec((1, 1), lambda v: (0, 0)),
        scratch_shapes=[pltpu.VMEM((SD, H), F32),
                        pltpu.VMEM((MS, 128), F32),
                        pltpu.SemaphoreType.DMA],
        compiler_params=pltpu.CompilerParams(
            dimension_semantics=("arbitrary",),
            vmem_limit_bytes=56 * 1024 * 1024),
        name="coref_pairs",
    )(x, pw_w2, pw_w3, a, b, b1, b2, b3, sg, pw_w1)
    return loss.reshape(())


# speedup vs baseline: 1.0893x; 1.0147x over previous
import jax
import jax.numpy as jnp
from jax.experimental import pallas as pl
from jax.experimental.pallas import tpu as pltpu

NEG = -1e10
F32 = jnp.float32
N_DOC, MS, W, BH, H, ED = 8, 16, 10, 768, 1024, 20
NS = N_DOC * MS
SD = 2 * BH + BH + ED
F_, R, D = 64, 36, 1024


def _prologue_kernel(se_ref, cont_ref, width_ref, doc_ref, img_ref,
                     aw1_ref, ab1_ref, aw2_ref, ab2_ref, wemb_ref, w1_any,
                     x_ref, a_ref, b_ref, sg_ref, w1ab_vmem, sem):
    cp = pltpu.make_async_copy(w1_any.at[pl.ds(0, 2 * SD), :], w1ab_vmem, sem)
    cp.start()
    width = width_ref[...]
    score_cols = []
    for t in range(W):
        ct = cont_ref[:, t * BH:(t + 1) * BH]
        h = jnp.maximum(
            jnp.dot(ct, aw1_ref[...], preferred_element_type=F32) + ab1_ref[...],
            0.0)
        score_cols.append(jnp.dot(h, aw2_ref[...], preferred_element_type=F32)
                          + ab2_ref[...])
    scores = jnp.concatenate(score_cols, axis=1)
    tok = jax.lax.broadcasted_iota(jnp.int32, (NS, W), 1).astype(F32)
    scores = jnp.where(tok < width, scores, NEG)
    m = jnp.max(scores, axis=1, keepdims=True)
    e = jnp.exp(scores - m)
    attn = e / jnp.sum(e, axis=1, keepdims=True)
    weighted = jnp.zeros((NS, BH), F32)
    for t in range(W):
        weighted = weighted + attn[:, t:t + 1] * cont_ref[:, t * BH:(t + 1) * BH]
    wclip = jnp.minimum(width, 4.0)
    wfeat = jnp.zeros((NS, ED), F32)
    for k in range(5):
        wfeat = wfeat + jnp.where(wclip == k, 1.0, 0.0) * wemb_ref[k:k + 1, :]
    x = jnp.concatenate([se_ref[...], weighted, wfeat], axis=1)
    x_ref[...] = x

    doc = doc_ref[...]
    sg_cols = []
    for v in range(N_DOC):
        imgv = img_ref[v * R:(v + 1) * R, :]
        att = jax.lax.dot_general(doc, imgv, (((1,), (1,)), ((), ())),
                                  preferred_element_type=F32)
        att = att.reshape(N_DOC, F_, R)
        att = jnp.where(att != 0.0, att, NEG)
        m1 = jnp.max(att, axis=2, keepdims=True)
        e1 = jnp.exp(att - m1)
        aw1 = e1 / jnp.sum(e1, axis=2, keepdims=True)
        s1 = jnp.sum(aw1 * att, axis=(1, 2))
        m2 = jnp.max(att, axis=1, keepdims=True)
        e2 = jnp.exp(att - m2)
        aw2 = e2 / jnp.sum(e2, axis=1, keepdims=True)
        s2 = jnp.sum(aw2 * att, axis=(1, 2))
        sg_cols.append((s1 + s2).reshape(N_DOC, 1))
    sg_ref[...] = jnp.concatenate(sg_cols, axis=1)

    cp.wait()
    a_ref[...] = jnp.dot(x, w1ab_vmem[0:SD, :], preferred_element_type=F32)
    xs = jnp.concatenate([jnp.zeros((NS, 4), F32), x], axis=1)
    b_ref[...] = jnp.dot(xs, w1ab_vmem[SD - 4:2 * SD, :],
                         preferred_element_type=F32)


def _pair_kernel(xf_ref, w2_ref, w3_ref, a_ref, b_ref,
                 b1_ref, b2_ref, b3_ref, sg_ref, w1_any, loss_ref,
                 w1c_ref, sc_ref, sem):
    v = pl.program_id(0)

    @pl.when(v == 0)
    def _():
        cp = pltpu.make_async_copy(w1_any.at[pl.ds(2 * SD, SD), :],
                                   w1c_ref, sem)
        cp.start()
        cp.wait()
        sc_ref[...] = jnp.zeros_like(sc_ref)

    lane16 = jax.lax.broadcasted_iota(jnp.int32, (1, MS), 1)

    def body(j, ts):
        q = v * MS + j
        xq = xf_ref[pl.ds(q, 1), :]
        z = jnp.dot(xf_ref[...] * xq, w1c_ref[...],
                    preferred_element_type=F32)
        h1 = jnp.maximum(z + a_ref[...] + b_ref[pl.ds(q, 1), :]
                         + b1_ref[...], 0.0)
        h2 = jnp.maximum(jnp.dot(h1, w2_ref[...],
                                 preferred_element_type=F32) + b2_ref[...], 0.0)
        col = jnp.dot(h2, w3_ref[...], preferred_element_type=F32) \
            + b3_ref[...]
        onehot = jnp.where(lane16 == j, 1.0, 0.0)
        return ts + col * onehot

    ts = jnp.zeros((NS, MS), F32)
    for j in range(MS):
        ts = body(j, ts)
    rm = jnp.mean(ts, axis=1, keepdims=True)
    max1 = jnp.max(rm.reshape(N_DOC, MS, 1), axis=1)
    cm = jnp.mean(ts.reshape(N_DOC, MS, MS), axis=1)
    max2 = jnp.max(cm, axis=1, keepdims=True)
    both = jnp.concatenate([max1, max2], axis=0)
    onehot_v = jnp.where(
        jax.lax.broadcasted_iota(jnp.int32, (1, 128), 1) == v, 1.0, 0.0)
    sc_ref[...] = sc_ref[...] + both * onehot_v

    @pl.when(v == N_DOC - 1)
    def _():
        sc = 0.5 * (sc_ref[0:N_DOC, 0:N_DOC]
                    + sc_ref[N_DOC:2 * N_DOC, 0:N_DOC])
        sg = sg_ref[...]

        def rowsoftmax(m_):
            mx = jnp.max(m_, axis=1, keepdims=True)
            e = jnp.exp(m_ - mx)
            return e / jnp.sum(e, axis=1, keepdims=True)

        mg = rowsoftmax(sg)
        mgT = rowsoftmax(sg.T)
        mc = rowsoftmax(sc)
        t1 = jnp.log(jnp.sum(mg * mc, axis=1, keepdims=True))
        t2 = jnp.log(jnp.sum(mgT * mc, axis=1, keepdims=True))
        loss = -(jnp.sum(t1) + jnp.sum(t2)) / float(N_DOC)
        loss_ref[...] = jnp.broadcast_to(loss, (1, 1))


def kernel(doc_embeddings, image_embeddings, text_mask, image_mask,
           start_end_embeddings, continuous_embeddings, width, span_mask,
           attn_w1, attn_b1, attn_w2, attn_b2, width_emb,
           pw_w1, pw_b1, pw_w2, pw_b2, pw_w3, pw_b3):
    se = start_end_embeddings.reshape(NS, 2 * BH)
    cont = continuous_embeddings.reshape(NS, W * BH)
    wid = width.astype(F32).reshape(NS, 1)
    docR = doc_embeddings.reshape(N_DOC * F_, D)
    imgR = image_embeddings.reshape(N_DOC * R, D)
    ab1 = attn_b1.reshape(1, H)
    ab2 = attn_b2.reshape(1, 1)
    b1 = pw_b1.reshape(1, H)
    b2 = pw_b2.reshape(1, H)
    b3 = pw_b3.reshape(1, 1)

    x, a, b, sg = pl.pallas_call(
        _prologue_kernel,
        out_shape=[
            jax.ShapeDtypeStruct((NS, SD), F32),
            jax.ShapeDtypeStruct((NS, H), F32),
            jax.ShapeDtypeStruct((NS, H), F32),
            jax.ShapeDtypeStruct((N_DOC, N_DOC), F32),
        ],
        in_specs=[pl.BlockSpec()] * 10 + [pl.BlockSpec(memory_space=pl.ANY)],
        scratch_shapes=[pltpu.VMEM((2 * SD, H), F32), pltpu.SemaphoreType.DMA],
        compiler_params=pltpu.CompilerParams(
            vmem_limit_bytes=56 * 1024 * 1024),
        name="coref_prologue",
    )(se, cont, wid, docR, imgR, attn_w1, ab1, attn_w2, ab2, width_emb, pw_w1)

    loss = pl.pallas_call(
        _pair_kernel,
        out_shape=jax.ShapeDtypeStruct((1, 1), F32),
        grid=(8,),
        in_specs=[
            pl.BlockSpec((NS, SD), lambda v: (0, 0)),
            pl.BlockSpec((H, H), lambda v: (0, 0)),
            pl.BlockSpec((H, 1), lambda v: (0, 0)),
            pl.BlockSpec((NS, H), lambda v: (0, 0)),
            pl.BlockSpec((NS, H), lambda v: (0, 0)),
            pl.BlockSpec((1, H), lambda v: (0, 0)),
            pl.BlockSpec((1, H), lambda v: (0, 0)),
            pl.BlockSpec((1, 1), lambda v: (0, 0)),
            pl.BlockSpec((N_DOC, N_DOC), lambda v: (0, 0)),
            pl.BlockSpec(memory_space=pl.ANY),
        ],
        out_specs=pl.BlockSpec((1, 1), lambda v: (0, 0)),
        scratch_shapes=[pltpu.VMEM((SD, H), F32),
                        pltpu.VMEM((MS, 128), F32),
                        pltpu.SemaphoreType.DMA],
        compiler_params=pltpu.CompilerParams(
            dimension_semantics=("arbitrary",),
            vmem_limit_bytes=56 * 1024 * 1024),
        name="coref_pairs",
    )(x, pw_w2, pw_w3, a, b, b1, b2, b3, sg, pw_w1)
    return loss.reshape(())
